# baseline (speedup 1.0000x reference)
import functools

import jax
import jax.numpy as jnp
import numpy as np
from jax.experimental import pallas as pl

N_RAYS = 2048
N_SAMPLES = 128
GRID = 96
COLOR_DIM = 12
WIDTH = 192
POS_PE = 5
VIEW_PE = 4
NEAR = 0.2
FAR = 2.0
XYZ_MIN = -1.0
XYZ_MAX = 1.0
STEPSIZE = 0.5
VOXEL_SIZE = (XYZ_MAX - XYZ_MIN) / GRID
MASKCACHE_THRES = 1e-3
FASTCOLOR_THRES = 1e-4
SMOOTH_K = 3
SMOOTH_SIGMA = 1.0
FEAT_DIM = 3 + 3 * POS_PE * 2 + 3 * VIEW_PE * 3 + 3

RAYS_PER_TILE = 16
PTS_PER_TILE = RAYS_PER_TILE * N_SAMPLES
N_TILES = N_RAYS // RAYS_PER_TILE


def _gauss_kernel():
    ax = np.arange(SMOOTH_K) - (SMOOTH_K - 1) / 2.0
    g = np.exp(-(ax ** 2) / (2.0 * SMOOTH_SIGMA ** 2))
    k3 = g[:, None, None] * g[None, :, None] * g[None, None, :]
    return jnp.asarray(k3 / k3.sum(), dtype=jnp.float32)


def _smooth(sdf):
    kern = _gauss_kernel()
    out = jax.lax.conv_general_dilated(sdf[None, None], kern[None, None], (1, 1, 1), 'SAME',
                                       dimension_numbers=('NCDHW', 'OIDHW', 'NCDHW'))
    return out[0, 0]


def _sdf_gradient(sdf):
    gx = jnp.zeros_like(sdf).at[1:-1, :, :].set((sdf[2:, :, :] - sdf[:-2, :, :]) / (2 * VOXEL_SIZE))
    gy = jnp.zeros_like(sdf).at[:, 1:-1, :].set((sdf[:, 2:, :] - sdf[:, :-2, :]) / (2 * VOXEL_SIZE))
    gz = jnp.zeros_like(sdf).at[:, :, 1:-1].set((sdf[:, :, 2:] - sdf[:, :, :-2]) / (2 * VOXEL_SIZE))
    return jnp.stack([gx, gy, gz], 0)


def _trilinear(grid, pts01):
    C = grid.shape[0]
    c = pts01 * float(GRID - 1)
    c0 = jnp.clip(jnp.floor(c).astype(jnp.int32), 0, GRID - 2)
    f = c - c0.astype(jnp.float32)
    out = jnp.zeros((pts01.shape[0], C), dtype=grid.dtype)
    for dx in (0, 1):
        wx = f[:, 0] if dx else 1.0 - f[:, 0]
        for dy in (0, 1):
            wy = f[:, 1] if dy else 1.0 - f[:, 1]
            for dz in (0, 1):
                wz = f[:, 2] if dz else 1.0 - f[:, 2]
                v = grid[:, c0[:, 0] + dx, c0[:, 1] + dy, c0[:, 2] + dz]
                out = out + (wx * wy * wz)[:, None] * v.T
    return out


HI = jax.lax.Precision.HIGHEST


def _tc_kernel(alpha_ref, feat_ref, offc_ref, emoc_ref, em_ref,
               ow0c_ref, ow0f_ref, ob0_ref, ow1_ref, ob1_ref, ow2_ref, ob2_ref,
               ew0c_ref, ew0f_ref, eb0_ref, ew1_ref, eb1_ref, ew2_ref, eb2_ref,
               ainv_ref, cumw_ref, rgbm_ref):
    R, S, P = RAYS_PER_TILE, N_SAMPLES, PTS_PER_TILE
    a = alpha_ref[0]

    row = jax.lax.broadcasted_iota(jnp.int32, (S, S), 0)
    col = jax.lax.broadcasted_iota(jnp.int32, (S, S), 1)
    m_strict = (row < col).astype(jnp.float32)

    s1 = jnp.log(1.0 - a + 1e-10)
    ts1 = jnp.exp(jax.lax.dot_general(s1, m_strict, (((1,), (0,)), ((), ())), precision=HI))
    w1 = a * ts1
    a2 = jnp.where(w1 > FASTCOLOR_THRES, a, 0.0)
    s2 = jnp.log(1.0 - a2 + 1e-10)
    ts2 = jnp.exp(jax.lax.dot_general(s2, m_strict, (((1,), (0,)), ((), ())), precision=HI))
    w = a2 * ts2
    ainv_last = jnp.exp(jnp.sum(s2, axis=1, keepdims=True))

    pid = jax.lax.broadcasted_iota(jnp.int32, (P, R), 0)
    rid = jax.lax.broadcasted_iota(jnp.int32, (P, R), 1)
    ind = (pid // S == rid).astype(jnp.float32)
    w_big = jax.lax.dot_general(ind, w, (((1,), (0,)), ((), ())), precision=HI)
    ps = jax.lax.broadcasted_iota(jnp.int32, (P, S), 0)
    ss = jax.lax.broadcasted_iota(jnp.int32, (P, S), 1)
    sel = (ps % S == ss).astype(jnp.float32)
    w_col = jnp.sum(w_big * sel, axis=1, keepdims=True)

    feat = feat_ref[...]

    def head(cref, w0c, w0f, b0, wm1, b1, wm2, b2):
        h = jax.lax.dot_general(cref[...], w0c[...], (((1,), (0,)), ((), ())), precision=HI)
        h = h + jax.lax.dot_general(feat, w0f[...], (((1,), (0,)), ((), ())), precision=HI)
        h = jax.nn.relu(h + b0[...])
        h = jax.nn.relu(jax.lax.dot_general(h, wm1[...], (((1,), (0,)), ((), ())), precision=HI) + b1[...])
        o = jax.lax.dot_general(h, wm2[...], (((1,), (0,)), ((), ())), precision=HI) + b2[...]
        return jax.nn.sigmoid(o)

    rgb_off = head(offc_ref, ow0c_ref, ow0f_ref, ob0_ref, ow1_ref, ob1_ref, ow2_ref, ob2_ref)
    rgb_emo = head(emoc_ref, ew0c_ref, ew0f_ref, eb0_ref, ew1_ref, eb1_ref, ew2_ref, eb2_ref)
    rgb = em_ref[...] * rgb_emo + rgb_off

    rgb_w = rgb * w_col
    rgbm = jax.lax.dot_general(ind, rgb_w, (((0,), (0,)), ((), ())), precision=HI)
    cumw = jax.lax.dot_general(ind, w_col, (((0,), (0,)), ((), ())), precision=HI)

    ainv_ref[0] = ainv_last
    cumw_ref[0] = cumw
    rgbm_ref[0] = rgbm


def kernel(rays_o, rays_d, viewdirs, em_modes, sdf_grid, mask_grid, off_grid, emo_grid,
           off_w0, off_b0, off_w1, off_b1, off_w2, off_b2,
           emo_w0, emo_b0, emo_w1, emo_b1, emo_w2, emo_b2, s_val):
    s_val = jnp.asarray(s_val, jnp.float32)
    t = NEAR + (FAR - NEAR) * (jnp.arange(N_SAMPLES, dtype=jnp.float32) + 0.5) / N_SAMPLES
    pts = (rays_o[:, None, :] + rays_d[:, None, :] * t[None, :, None]).reshape(-1, 3)
    ray_id = jnp.repeat(jnp.arange(N_RAYS), N_SAMPLES)
    pts01 = jnp.clip((pts - XYZ_MIN) / (XYZ_MAX - XYZ_MIN), 0.0, 1.0)
    valid = _trilinear(mask_grid[None], pts01)[:, 0] > MASKCACHE_THRES
    sdf_s = _smooth(sdf_grid)
    sdf = _trilinear(sdf_s[None], pts01)[:, 0]
    gradient = _trilinear(_sdf_gradient(sdf_s), pts01)
    dist = STEPSIZE * VOXEL_SIZE
    dirs = viewdirs[ray_id]
    true_cos = jnp.sum(dirs * gradient, -1)
    iter_cos = -(jax.nn.relu(-true_cos * 0.5 + 0.5) * 0.5 + jax.nn.relu(-true_cos) * 0.5)
    prev_cdf = jax.nn.sigmoid((sdf - iter_cos * dist * 0.5) * s_val)
    next_cdf = jax.nn.sigmoid((sdf + iter_cos * dist * 0.5) * s_val)
    alpha = jnp.clip((prev_cdf - next_cdf + 1e-5) / (prev_cdf + 1e-5), 0.0, 1.0)
    alpha = jnp.where(valid, alpha, 0.0)

    posfreq = jnp.asarray([2.0 ** i for i in range(POS_PE)], jnp.float32)
    viewfreq = jnp.asarray([2.0 ** i for i in range(VIEW_PE)], jnp.float32)
    xyz_emb = (pts01[..., None] * posfreq).reshape(pts01.shape[0], -1)
    view_emb = (viewdirs[..., None] * viewfreq).reshape(N_RAYS, -1)
    normal = gradient / (jnp.linalg.norm(gradient, axis=-1, keepdims=True) + 1e-5)
    rgb_feat = jnp.concatenate([pts01, jnp.sin(xyz_emb), jnp.cos(xyz_emb),
                                view_emb[ray_id], jnp.sin(view_emb)[ray_id], jnp.cos(view_emb)[ray_id],
                                normal], -1)
    off_c = _trilinear(off_grid, pts01)
    emo_c = _trilinear(emo_grid, pts01)
    em_col = (em_modes == 1).astype(jnp.float32)[ray_id][:, None]

    alpha_rs = alpha.reshape(N_TILES, RAYS_PER_TILE, N_SAMPLES)

    R, P = RAYS_PER_TILE, PTS_PER_TILE
    row_spec = lambda d: pl.BlockSpec((P, d), lambda i: (i, 0))
    full_spec = lambda shp: pl.BlockSpec(shp, lambda i: tuple(0 for _ in shp))

    out_shapes = (
        jax.ShapeDtypeStruct((N_TILES, R, 1), jnp.float32),
        jax.ShapeDtypeStruct((N_TILES, R, 1), jnp.float32),
        jax.ShapeDtypeStruct((N_TILES, R, 3), jnp.float32),
    )
    ainv, cumw, rgbm = pl.pallas_call(
        _tc_kernel,
        grid=(N_TILES,),
        in_specs=[
            pl.BlockSpec((1, R, N_SAMPLES), lambda i: (i, 0, 0)),
            row_spec(FEAT_DIM),
            row_spec(COLOR_DIM),
            row_spec(COLOR_DIM),
            row_spec(1),
            full_spec(off_w0[:COLOR_DIM].shape), full_spec(off_w0[COLOR_DIM:].shape),
            full_spec((1, WIDTH)), full_spec(off_w1.shape), full_spec((1, WIDTH)),
            full_spec(off_w2.shape), full_spec((1, 3)),
            full_spec(emo_w0[:COLOR_DIM].shape), full_spec(emo_w0[COLOR_DIM:].shape),
            full_spec((1, WIDTH)), full_spec(emo_w1.shape), full_spec((1, WIDTH)),
            full_spec(emo_w2.shape), full_spec((1, 3)),
        ],
        out_specs=(
            pl.BlockSpec((1, R, 1), lambda i: (i, 0, 0)),
            pl.BlockSpec((1, R, 1), lambda i: (i, 0, 0)),
            pl.BlockSpec((1, R, 3), lambda i: (i, 0, 0)),
        ),
        out_shape=out_shapes,
    )(alpha_rs, rgb_feat, off_c, emo_c, em_col,
      off_w0[:COLOR_DIM], off_w0[COLOR_DIM:], off_b0[None], off_w1, off_b1[None], off_w2, off_b2[None],
      emo_w0[:COLOR_DIM], emo_w0[COLOR_DIM:], emo_b0[None], emo_w1, emo_b1[None], emo_w2, emo_b2[None])

    alphainv_last = ainv.reshape(N_RAYS)
    cum_weights = cumw.reshape(N_RAYS, 1)
    rgb_marched = rgbm.reshape(N_RAYS, 3)
    return alphainv_last, 1.0 - cum_weights, rgb_marched


# speedup vs baseline: 1.0723x; 1.0723x over previous
import functools

import jax
import jax.numpy as jnp
import numpy as np
from jax.experimental import pallas as pl

N_RAYS = 2048
N_SAMPLES = 128
GRID = 96
COLOR_DIM = 12
WIDTH = 192
POS_PE = 5
VIEW_PE = 4
NEAR = 0.2
FAR = 2.0
XYZ_MIN = -1.0
XYZ_MAX = 1.0
STEPSIZE = 0.5
VOXEL_SIZE = (XYZ_MAX - XYZ_MIN) / GRID
MASKCACHE_THRES = 1e-3
FASTCOLOR_THRES = 1e-4
SMOOTH_K = 3
SMOOTH_SIGMA = 1.0
FEAT_DIM = 3 + 3 * POS_PE * 2 + 3 * VIEW_PE * 3 + 3

RAYS_PER_TILE = 16
PTS_PER_TILE = RAYS_PER_TILE * N_SAMPLES
N_TILES = N_RAYS // RAYS_PER_TILE


def _gauss_kernel():
    ax = np.arange(SMOOTH_K) - (SMOOTH_K - 1) / 2.0
    g = np.exp(-(ax ** 2) / (2.0 * SMOOTH_SIGMA ** 2))
    k3 = g[:, None, None] * g[None, :, None] * g[None, None, :]
    return jnp.asarray(k3 / k3.sum(), dtype=jnp.float32)


def _smooth(sdf):
    kern = _gauss_kernel()
    out = jax.lax.conv_general_dilated(sdf[None, None], kern[None, None], (1, 1, 1), 'SAME',
                                       dimension_numbers=('NCDHW', 'OIDHW', 'NCDHW'))
    return out[0, 0]


def _sdf_gradient(sdf):
    gx = jnp.zeros_like(sdf).at[1:-1, :, :].set((sdf[2:, :, :] - sdf[:-2, :, :]) / (2 * VOXEL_SIZE))
    gy = jnp.zeros_like(sdf).at[:, 1:-1, :].set((sdf[:, 2:, :] - sdf[:, :-2, :]) / (2 * VOXEL_SIZE))
    gz = jnp.zeros_like(sdf).at[:, :, 1:-1].set((sdf[:, :, 2:] - sdf[:, :, :-2]) / (2 * VOXEL_SIZE))
    return jnp.stack([gx, gy, gz], 0)


def _trilinear(grid, pts01):
    C = grid.shape[0]
    c = pts01 * float(GRID - 1)
    c0 = jnp.clip(jnp.floor(c).astype(jnp.int32), 0, GRID - 2)
    f = c - c0.astype(jnp.float32)
    out = jnp.zeros((pts01.shape[0], C), dtype=grid.dtype)
    for dx in (0, 1):
        wx = f[:, 0] if dx else 1.0 - f[:, 0]
        for dy in (0, 1):
            wy = f[:, 1] if dy else 1.0 - f[:, 1]
            for dz in (0, 1):
                wz = f[:, 2] if dz else 1.0 - f[:, 2]
                v = grid[:, c0[:, 0] + dx, c0[:, 1] + dy, c0[:, 2] + dz]
                out = out + (wx * wy * wz)[:, None] * v.T
    return out


HI = jax.lax.Precision.HIGHEST


def _tc_kernel(alpha_ref, feat_ref, offc_ref, emoc_ref, em_ref,
               ow0c_ref, ow0f_ref, ob0_ref, ow1_ref, ob1_ref, ow2_ref, ob2_ref,
               ew0c_ref, ew0f_ref, eb0_ref, ew1_ref, eb1_ref, ew2_ref, eb2_ref,
               ainv_ref, cumw_ref, rgbm_ref):
    R, S, P = RAYS_PER_TILE, N_SAMPLES, PTS_PER_TILE
    a = alpha_ref[0]

    row = jax.lax.broadcasted_iota(jnp.int32, (S, S), 0)
    col = jax.lax.broadcasted_iota(jnp.int32, (S, S), 1)
    m_strict = (row < col).astype(jnp.float32)

    s1 = jnp.log(1.0 - a + 1e-10)
    ts1 = jnp.exp(jax.lax.dot_general(s1, m_strict, (((1,), (0,)), ((), ())), precision=HI))
    w1 = a * ts1
    a2 = jnp.where(w1 > FASTCOLOR_THRES, a, 0.0)
    s2 = jnp.log(1.0 - a2 + 1e-10)
    ts2 = jnp.exp(jax.lax.dot_general(s2, m_strict, (((1,), (0,)), ((), ())), precision=HI))
    w = a2 * ts2
    ainv_last = jnp.exp(jnp.sum(s2, axis=1, keepdims=True))

    pid = jax.lax.broadcasted_iota(jnp.int32, (P, R), 0)
    rid = jax.lax.broadcasted_iota(jnp.int32, (P, R), 1)
    ind = (pid // S == rid).astype(jnp.float32)
    w_big = jax.lax.dot_general(ind, w, (((1,), (0,)), ((), ())), precision=HI)
    ps = jax.lax.broadcasted_iota(jnp.int32, (P, S), 0)
    ss = jax.lax.broadcasted_iota(jnp.int32, (P, S), 1)
    sel = (ps % S == ss).astype(jnp.float32)
    w_col = jnp.sum(w_big * sel, axis=1, keepdims=True)

    feat = feat_ref[...]

    MP = jax.lax.Precision.DEFAULT

    def head(cref, w0c, w0f, b0, wm1, b1, wm2, b2):
        h = jax.lax.dot_general(cref[...], w0c[...], (((1,), (0,)), ((), ())), precision=MP)
        h = h + jax.lax.dot_general(feat, w0f[...], (((1,), (0,)), ((), ())), precision=MP)
        h = jax.nn.relu(h + b0[...])
        h = jax.nn.relu(jax.lax.dot_general(h, wm1[...], (((1,), (0,)), ((), ())), precision=MP) + b1[...])
        o = jax.lax.dot_general(h, wm2[...], (((1,), (0,)), ((), ())), precision=MP) + b2[...]
        return jax.nn.sigmoid(o)

    rgb_off = head(offc_ref, ow0c_ref, ow0f_ref, ob0_ref, ow1_ref, ob1_ref, ow2_ref, ob2_ref)
    rgb_emo = head(emoc_ref, ew0c_ref, ew0f_ref, eb0_ref, ew1_ref, eb1_ref, ew2_ref, eb2_ref)
    rgb = em_ref[...] * rgb_emo + rgb_off

    rgb_w = rgb * w_col
    rgbm = jax.lax.dot_general(ind, rgb_w, (((0,), (0,)), ((), ())), precision=HI)
    cumw = jax.lax.dot_general(ind, w_col, (((0,), (0,)), ((), ())), precision=HI)

    ainv_ref[0] = ainv_last
    cumw_ref[0] = cumw
    rgbm_ref[0] = rgbm


def kernel(rays_o, rays_d, viewdirs, em_modes, sdf_grid, mask_grid, off_grid, emo_grid,
           off_w0, off_b0, off_w1, off_b1, off_w2, off_b2,
           emo_w0, emo_b0, emo_w1, emo_b1, emo_w2, emo_b2, s_val):
    s_val = jnp.asarray(s_val, jnp.float32)
    t = NEAR + (FAR - NEAR) * (jnp.arange(N_SAMPLES, dtype=jnp.float32) + 0.5) / N_SAMPLES
    pts = (rays_o[:, None, :] + rays_d[:, None, :] * t[None, :, None]).reshape(-1, 3)
    ray_id = jnp.repeat(jnp.arange(N_RAYS), N_SAMPLES)
    pts01 = jnp.clip((pts - XYZ_MIN) / (XYZ_MAX - XYZ_MIN), 0.0, 1.0)
    valid = _trilinear(mask_grid[None], pts01)[:, 0] > MASKCACHE_THRES
    sdf_s = _smooth(sdf_grid)
    sdf = _trilinear(sdf_s[None], pts01)[:, 0]
    gradient = _trilinear(_sdf_gradient(sdf_s), pts01)
    dist = STEPSIZE * VOXEL_SIZE
    dirs = viewdirs[ray_id]
    true_cos = jnp.sum(dirs * gradient, -1)
    iter_cos = -(jax.nn.relu(-true_cos * 0.5 + 0.5) * 0.5 + jax.nn.relu(-true_cos) * 0.5)
    prev_cdf = jax.nn.sigmoid((sdf - iter_cos * dist * 0.5) * s_val)
    next_cdf = jax.nn.sigmoid((sdf + iter_cos * dist * 0.5) * s_val)
    alpha = jnp.clip((prev_cdf - next_cdf + 1e-5) / (prev_cdf + 1e-5), 0.0, 1.0)
    alpha = jnp.where(valid, alpha, 0.0)

    posfreq = jnp.asarray([2.0 ** i for i in range(POS_PE)], jnp.float32)
    viewfreq = jnp.asarray([2.0 ** i for i in range(VIEW_PE)], jnp.float32)
    xyz_emb = (pts01[..., None] * posfreq).reshape(pts01.shape[0], -1)
    view_emb = (viewdirs[..., None] * viewfreq).reshape(N_RAYS, -1)
    normal = gradient / (jnp.linalg.norm(gradient, axis=-1, keepdims=True) + 1e-5)
    rgb_feat = jnp.concatenate([pts01, jnp.sin(xyz_emb), jnp.cos(xyz_emb),
                                view_emb[ray_id], jnp.sin(view_emb)[ray_id], jnp.cos(view_emb)[ray_id],
                                normal], -1)
    off_c = _trilinear(off_grid, pts01)
    emo_c = _trilinear(emo_grid, pts01)
    em_col = (em_modes == 1).astype(jnp.float32)[ray_id][:, None]

    alpha_rs = alpha.reshape(N_TILES, RAYS_PER_TILE, N_SAMPLES)

    R, P = RAYS_PER_TILE, PTS_PER_TILE
    row_spec = lambda d: pl.BlockSpec((P, d), lambda i: (i, 0))
    full_spec = lambda shp: pl.BlockSpec(shp, lambda i: tuple(0 for _ in shp))

    out_shapes = (
        jax.ShapeDtypeStruct((N_TILES, R, 1), jnp.float32),
        jax.ShapeDtypeStruct((N_TILES, R, 1), jnp.float32),
        jax.ShapeDtypeStruct((N_TILES, R, 3), jnp.float32),
    )
    ainv, cumw, rgbm = pl.pallas_call(
        _tc_kernel,
        grid=(N_TILES,),
        in_specs=[
            pl.BlockSpec((1, R, N_SAMPLES), lambda i: (i, 0, 0)),
            row_spec(FEAT_DIM),
            row_spec(COLOR_DIM),
            row_spec(COLOR_DIM),
            row_spec(1),
            full_spec(off_w0[:COLOR_DIM].shape), full_spec(off_w0[COLOR_DIM:].shape),
            full_spec((1, WIDTH)), full_spec(off_w1.shape), full_spec((1, WIDTH)),
            full_spec(off_w2.shape), full_spec((1, 3)),
            full_spec(emo_w0[:COLOR_DIM].shape), full_spec(emo_w0[COLOR_DIM:].shape),
            full_spec((1, WIDTH)), full_spec(emo_w1.shape), full_spec((1, WIDTH)),
            full_spec(emo_w2.shape), full_spec((1, 3)),
        ],
        out_specs=(
            pl.BlockSpec((1, R, 1), lambda i: (i, 0, 0)),
            pl.BlockSpec((1, R, 1), lambda i: (i, 0, 0)),
            pl.BlockSpec((1, R, 3), lambda i: (i, 0, 0)),
        ),
        out_shape=out_shapes,
    )(alpha_rs, rgb_feat, off_c, emo_c, em_col,
      off_w0[:COLOR_DIM], off_w0[COLOR_DIM:], off_b0[None], off_w1, off_b1[None], off_w2, off_b2[None],
      emo_w0[:COLOR_DIM], emo_w0[COLOR_DIM:], emo_b0[None], emo_w1, emo_b1[None], emo_w2, emo_b2[None])

    alphainv_last = ainv.reshape(N_RAYS)
    cum_weights = cumw.reshape(N_RAYS, 1)
    rgb_marched = rgbm.reshape(N_RAYS, 3)
    return alphainv_last, 1.0 - cum_weights, rgb_marched


# speedup vs baseline: 1.1180x; 1.0426x over previous
import functools

import jax
import jax.numpy as jnp
import numpy as np
from jax.experimental import pallas as pl
from jax.experimental.pallas import tpu as pltpu
from jax.experimental.pallas import tpu_sc as plsc

N_RAYS = 2048
N_SAMPLES = 128
GRID = 96
COLOR_DIM = 12
WIDTH = 192
POS_PE = 5
VIEW_PE = 4
NEAR = 0.2
FAR = 2.0
XYZ_MIN = -1.0
XYZ_MAX = 1.0
STEPSIZE = 0.5
VOXEL_SIZE = (XYZ_MAX - XYZ_MIN) / GRID
MASKCACHE_THRES = 1e-3
FASTCOLOR_THRES = 1e-4
SMOOTH_K = 3
SMOOTH_SIGMA = 1.0
FEAT_DIM = 3 + 3 * POS_PE * 2 + 3 * VIEW_PE * 3 + 3

N_PTS = N_RAYS * N_SAMPLES
TABLE_C = 32
ROW_F = 8 * TABLE_C
TABLE_V = GRID * GRID * GRID

SC_NC = 2
SC_NS = 16
SC_NW = SC_NC * SC_NS
B_TOTAL = N_PTS
B_PER_W = B_TOTAL // SC_NW
SC_CHUNK = 256
SC_NCHUNK = B_PER_W // SC_CHUNK

RAYS_PER_TILE = 32
PTS_PER_TILE = RAYS_PER_TILE * N_SAMPLES
N_TILES = N_RAYS // RAYS_PER_TILE

HI = jax.lax.Precision.HIGHEST
MP = jax.lax.Precision.DEFAULT


def _gauss_kernel():
    ax = np.arange(SMOOTH_K) - (SMOOTH_K - 1) / 2.0
    g = np.exp(-(ax ** 2) / (2.0 * SMOOTH_SIGMA ** 2))
    k3 = g[:, None, None] * g[None, :, None] * g[None, None, :]
    return jnp.asarray(k3 / k3.sum(), dtype=jnp.float32)


def _smooth(sdf):
    kern = _gauss_kernel()
    out = jax.lax.conv_general_dilated(sdf[None, None], kern[None, None], (1, 1, 1), 'SAME',
                                       dimension_numbers=('NCDHW', 'OIDHW', 'NCDHW'))
    return out[0, 0]


def _sdf_gradient(sdf):
    gx = jnp.zeros_like(sdf).at[1:-1, :, :].set((sdf[2:, :, :] - sdf[:-2, :, :]) / (2 * VOXEL_SIZE))
    gy = jnp.zeros_like(sdf).at[:, 1:-1, :].set((sdf[:, 2:, :] - sdf[:, :-2, :]) / (2 * VOXEL_SIZE))
    gz = jnp.zeros_like(sdf).at[:, :, 1:-1].set((sdf[:, :, 2:] - sdf[:, :, :-2]) / (2 * VOXEL_SIZE))
    return gx, gy, gz


def _sc_gather_rows(table, idx):
    mesh = plsc.VectorSubcoreMesh(core_axis_name="c", subcore_axis_name="s")

    @functools.partial(
        pl.kernel,
        mesh=mesh,
        out_type=jax.ShapeDtypeStruct((B_TOTAL, ROW_F), jnp.float32),
        scratch_types=[
            pltpu.VMEM((SC_CHUNK,), jnp.int32),
            pltpu.VMEM((SC_CHUNK, ROW_F), jnp.float32),
            pltpu.SemaphoreType.DMA,
        ],
    )
    def k(table_hbm, idx_hbm, out_hbm, idx_v, rows_v, sem):
        wid = jax.lax.axis_index("s") * SC_NC + jax.lax.axis_index("c")
        base0 = wid * B_PER_W

        def body(j, carry):
            base = base0 + j * SC_CHUNK
            pltpu.sync_copy(idx_hbm.at[pl.ds(base, SC_CHUNK)], idx_v)
            pltpu.async_copy(table_hbm.at[idx_v], rows_v, sem).wait()
            pltpu.sync_copy(rows_v, out_hbm.at[pl.ds(base, SC_CHUNK)])
            return carry

        jax.lax.fori_loop(0, SC_NCHUNK, body, 0)

    return k(table, idx)


def _comp_kernel(alpha_ref, w_ref, ainv_ref, cumw_ref):
    a = alpha_ref[...]
    S = N_SAMPLES
    row = jax.lax.broadcasted_iota(jnp.int32, (S, S), 0)
    col = jax.lax.broadcasted_iota(jnp.int32, (S, S), 1)
    m_strict = (row < col).astype(jnp.float32)

    s1 = jnp.log(1.0 - a + 1e-10)
    ts1 = jnp.exp(jax.lax.dot_general(s1, m_strict, (((1,), (0,)), ((), ())), precision=HI))
    w1 = a * ts1
    a2 = jnp.where(w1 > FASTCOLOR_THRES, a, 0.0)
    s2 = jnp.log(1.0 - a2 + 1e-10)
    ts2 = jnp.exp(jax.lax.dot_general(s2, m_strict, (((1,), (0,)), ((), ())), precision=HI))
    w = a2 * ts2
    w_ref[...] = w
    ainv_ref[...] = jnp.exp(jnp.sum(s2, axis=1, keepdims=True))
    cumw_ref[...] = jnp.sum(w, axis=1, keepdims=True)


def _mlp_kernel(feat_ref, offc_ref, emoc_ref, em_ref, wcol_ref,
                ow0c_ref, ow0f_ref, ob0_ref, ow1_ref, ob1_ref, ow2_ref, ob2_ref,
                ew0c_ref, ew0f_ref, eb0_ref, ew1_ref, eb1_ref, ew2_ref, eb2_ref,
                rgbm_ref):
    P, R = PTS_PER_TILE, RAYS_PER_TILE
    feat = feat_ref[...]

    def head(cref, w0c, w0f, b0, wm1, b1, wm2, b2):
        h = jax.lax.dot_general(cref[...], w0c[...], (((1,), (0,)), ((), ())), precision=MP)
        h = h + jax.lax.dot_general(feat, w0f[...], (((1,), (0,)), ((), ())), precision=MP)
        h = jax.nn.relu(h + b0[...])
        h = jax.nn.relu(jax.lax.dot_general(h, wm1[...], (((1,), (0,)), ((), ())), precision=MP) + b1[...])
        o = jax.lax.dot_general(h, wm2[...], (((1,), (0,)), ((), ())), precision=MP) + b2[...]
        return jax.nn.sigmoid(o)

    rgb_off = head(offc_ref, ow0c_ref, ow0f_ref, ob0_ref, ow1_ref, ob1_ref, ow2_ref, ob2_ref)
    rgb_emo = head(emoc_ref, ew0c_ref, ew0f_ref, eb0_ref, ew1_ref, eb1_ref, ew2_ref, eb2_ref)
    rgb = em_ref[...] * rgb_emo + rgb_off
    rgb_w = rgb * wcol_ref[...]

    pid = jax.lax.broadcasted_iota(jnp.int32, (P, R), 0)
    rid = jax.lax.broadcasted_iota(jnp.int32, (P, R), 1)
    ind = (pid // N_SAMPLES == rid).astype(jnp.float32)
    rgbm_ref[0] = jax.lax.dot_general(ind, rgb_w, (((0,), (0,)), ((), ())), precision=MP)


def kernel(rays_o, rays_d, viewdirs, em_modes, sdf_grid, mask_grid, off_grid, emo_grid,
           off_w0, off_b0, off_w1, off_b1, off_w2, off_b2,
           emo_w0, emo_b0, emo_w1, emo_b1, emo_w2, emo_b2, s_val):
    s_val = jnp.asarray(s_val, jnp.float32)
    t = NEAR + (FAR - NEAR) * (jnp.arange(N_SAMPLES, dtype=jnp.float32) + 0.5) / N_SAMPLES
    pts = (rays_o[:, None, :] + rays_d[:, None, :] * t[None, :, None]).reshape(-1, 3)
    ray_id = jnp.repeat(jnp.arange(N_RAYS), N_SAMPLES)
    pts01 = jnp.clip((pts - XYZ_MIN) / (XYZ_MAX - XYZ_MIN), 0.0, 1.0)

    sdf_s = _smooth(sdf_grid)
    gx, gy, gz = _sdf_gradient(sdf_s)
    chans = [sdf_s[None], gx[None], gy[None], gz[None], mask_grid[None], off_grid, emo_grid,
             jnp.zeros((TABLE_C - 29, GRID, GRID, GRID), jnp.float32)]
    chp = jnp.pad(jnp.concatenate(chans, 0), ((0, 0), (0, 1), (0, 1), (0, 1)))
    tp = chp.transpose(1, 2, 3, 0)
    corners = jnp.stack(
        [tp[dx:dx + GRID, dy:dy + GRID, dz:dz + GRID]
         for dx in (0, 1) for dy in (0, 1) for dz in (0, 1)], axis=3)
    table = corners.reshape(TABLE_V, ROW_F)

    c = pts01 * float(GRID - 1)
    c0 = jnp.clip(jnp.floor(c).astype(jnp.int32), 0, GRID - 2)
    f = c - c0.astype(jnp.float32)
    fx, fy, fz = f[:, 0], f[:, 1], f[:, 2]
    wx = jnp.stack([1.0 - fx, fx], 1)
    wy = jnp.stack([1.0 - fy, fy], 1)
    wz = jnp.stack([1.0 - fz, fz], 1)
    w8 = (wx[:, :, None, None] * wy[:, None, :, None] * wz[:, None, None, :]).reshape(-1, 8)
    idx = (c0[:, 0] * GRID + c0[:, 1]) * GRID + c0[:, 2]

    rows = _sc_gather_rows(table, idx)
    feats = jnp.einsum('nk,nkc->nc', w8, rows.reshape(N_PTS, 8, TABLE_C))

    sdf = feats[:, 0]
    gradient = feats[:, 1:4]
    valid = feats[:, 4] > MASKCACHE_THRES
    off_c = feats[:, 5:5 + COLOR_DIM]
    emo_c = feats[:, 17:17 + COLOR_DIM]

    dist = STEPSIZE * VOXEL_SIZE
    dirs = viewdirs[ray_id]
    true_cos = jnp.sum(dirs * gradient, -1)
    iter_cos = -(jax.nn.relu(-true_cos * 0.5 + 0.5) * 0.5 + jax.nn.relu(-true_cos) * 0.5)
    prev_cdf = jax.nn.sigmoid((sdf - iter_cos * dist * 0.5) * s_val)
    next_cdf = jax.nn.sigmoid((sdf + iter_cos * dist * 0.5) * s_val)
    alpha = jnp.clip((prev_cdf - next_cdf + 1e-5) / (prev_cdf + 1e-5), 0.0, 1.0)
    alpha = jnp.where(valid, alpha, 0.0).reshape(N_RAYS, N_SAMPLES)

    w_full, ainv, cumw = pl.pallas_call(
        _comp_kernel,
        out_shape=(
            jax.ShapeDtypeStruct((N_RAYS, N_SAMPLES), jnp.float32),
            jax.ShapeDtypeStruct((N_RAYS, 1), jnp.float32),
            jax.ShapeDtypeStruct((N_RAYS, 1), jnp.float32),
        ),
    )(alpha)
    wcol = w_full.reshape(N_PTS, 1)

    posfreq = jnp.asarray([2.0 ** i for i in range(POS_PE)], jnp.float32)
    viewfreq = jnp.asarray([2.0 ** i for i in range(VIEW_PE)], jnp.float32)
    xyz_emb = (pts01[..., None] * posfreq).reshape(pts01.shape[0], -1)
    view_emb = (viewdirs[..., None] * viewfreq).reshape(N_RAYS, -1)
    normal = gradient / (jnp.linalg.norm(gradient, axis=-1, keepdims=True) + 1e-5)
    rgb_feat = jnp.concatenate([pts01, jnp.sin(xyz_emb), jnp.cos(xyz_emb),
                                view_emb[ray_id], jnp.sin(view_emb)[ray_id], jnp.cos(view_emb)[ray_id],
                                normal], -1)
    em_col = (em_modes == 1).astype(jnp.float32)[ray_id][:, None]

    P = PTS_PER_TILE
    row_spec = lambda d: pl.BlockSpec((P, d), lambda i: (i, 0))
    full_spec = lambda shp: pl.BlockSpec(shp, lambda i: tuple(0 for _ in shp))

    rgbm = pl.pallas_call(
        _mlp_kernel,
        grid=(N_TILES,),
        in_specs=[
            row_spec(FEAT_DIM),
            row_spec(COLOR_DIM),
            row_spec(COLOR_DIM),
            row_spec(1),
            row_spec(1),
            full_spec(off_w0[:COLOR_DIM].shape), full_spec(off_w0[COLOR_DIM:].shape),
            full_spec((1, WIDTH)), full_spec(off_w1.shape), full_spec((1, WIDTH)),
            full_spec(off_w2.shape), full_spec((1, 3)),
            full_spec(emo_w0[:COLOR_DIM].shape), full_spec(emo_w0[COLOR_DIM:].shape),
            full_spec((1, WIDTH)), full_spec(emo_w1.shape), full_spec((1, WIDTH)),
            full_spec(emo_w2.shape), full_spec((1, 3)),
        ],
        out_specs=pl.BlockSpec((1, RAYS_PER_TILE, 3), lambda i: (i, 0, 0)),
        out_shape=jax.ShapeDtypeStruct((N_TILES, RAYS_PER_TILE, 3), jnp.float32),
    )(rgb_feat, off_c, emo_c, em_col, wcol,
      off_w0[:COLOR_DIM], off_w0[COLOR_DIM:], off_b0[None], off_w1, off_b1[None], off_w2, off_b2[None],
      emo_w0[:COLOR_DIM], emo_w0[COLOR_DIM:], emo_b0[None], emo_w1, emo_b1[None], emo_w2, emo_b2[None])

    alphainv_last = ainv.reshape(N_RAYS)
    cum_weights = cumw.reshape(N_RAYS, 1)
    rgb_marched = rgbm.reshape(N_RAYS, 3)
    return alphainv_last, 1.0 - cum_weights, rgb_marched
